# baseline (speedup 1.0000x reference)
import functools

import jax
import jax.numpy as jnp
from jax import lax
from jax.experimental import pallas as pl
from jax.experimental.pallas import tpu as pltpu
from jax.experimental.pallas import tpu_sc as plsc

_NC = 2
_NS = 16
_NW = _NC * _NS


def _sc_winpos_phase1(keys_pad, vals_pad, n_rows_pad, e_pad):
  per_w = e_pad // _NW
  nvec = per_w // 16
  mesh = plsc.VectorSubcoreMesh(core_axis_name="c", subcore_axis_name="s")

  slice_w = n_rows_pad // _NW

  @functools.partial(
      pl.kernel,
      out_type=jax.ShapeDtypeStruct((_NW, _NW, slice_w), jnp.int32),
      mesh=mesh,
      compiler_params=pltpu.CompilerParams(needs_layout_passes=False),
      scratch_types=[
          pltpu.VMEM((per_w,), jnp.int32),
          pltpu.VMEM((per_w,), jnp.int32),
          pltpu.VMEM((n_rows_pad,), jnp.int32),
          pltpu.SemaphoreType.DMA,
      ],
  )
  def k(keys_hbm, vals_hbm, out_hbm, keys_v, vals_v, lval, osem):
    wid = lax.axis_index("s") * _NC + lax.axis_index("c")
    base = wid * per_w
    pltpu.sync_copy(keys_hbm.at[pl.ds(base, per_w)], keys_v)
    pltpu.sync_copy(vals_hbm.at[pl.ds(base, per_w)], vals_v)

    minus1 = jnp.full((16,), -1, jnp.int32)

    def init_body(i, carry):
      for j in range(8):
        lval[pl.ds((i * 8 + j) * 16, 16)] = minus1
      return carry

    lax.fori_loop(0, n_rows_pad // (16 * 8), init_body, 0)

    lane = lax.iota(jnp.int32, 16)

    def scat_body(i, carry):
      keys = keys_v[pl.ds(i * 16, 16)]
      vals = vals_v[pl.ds(i * 16, 16)]
      for l in range(16):
        plsc.store_scatter(lval, [keys], vals, mask=lane == l)
      return carry

    lax.fori_loop(0, nvec, scat_body, 0)
    for s in range(_NW):
      pltpu.async_copy(lval.at[pl.ds(s * slice_w, slice_w)],
                       out_hbm.at[s, wid], osem)
    for s in range(_NW):
      pltpu.make_async_copy(lval.at[pl.ds(s * slice_w, slice_w)],
                            out_hbm.at[s, wid], osem).wait()

  return k(keys_pad, vals_pad)


def _sc_winpos_phase2(lval_all, n_rows_pad, n_cfg):
  per_w = n_rows_pad // _NW
  nv = per_w // 16
  mesh = plsc.VectorSubcoreMesh(core_axis_name="c", subcore_axis_name="s")

  @functools.partial(
      pl.kernel,
      out_type=(jax.ShapeDtypeStruct((n_rows_pad,), jnp.int32),
                jax.ShapeDtypeStruct((n_rows_pad,), jnp.int32)),
      mesh=mesh,
      scratch_types=[
          pltpu.VMEM((_NW, per_w), jnp.int32),
          pltpu.VMEM((per_w,), jnp.int32),
          pltpu.VMEM((per_w,), jnp.int32),
      ],
  )
  def k(lval_hbm, wp_hbm, vs_hbm, tbl_v, wp_v, vs_v):
    wid = lax.axis_index("s") * _NC + lax.axis_index("c")
    base = wid * per_w
    pltpu.sync_copy(lval_hbm.at[wid], tbl_v)

    lane = lax.iota(jnp.int32, 16)
    dummy_mask = 32767
    assert n_cfg > dummy_mask

    def max_body(i, carry):
      acc = tbl_v[0, pl.ds(i * 16, 16)]
      for t in range(1, _NW):
        cur = tbl_v[t, pl.ds(i * 16, 16)]
        acc = jnp.where(cur >= 0, cur, acc)
      wp_v[pl.ds(i * 16, 16)] = acc
      rows = base + i * 16 + lane
      vs_v[pl.ds(i * 16, 16)] = jnp.where(acc >= 0, acc, rows & dummy_mask)
      return carry

    lax.fori_loop(0, nv, max_body, 0)
    pltpu.sync_copy(wp_v, wp_hbm.at[pl.ds(base, per_w)])
    pltpu.sync_copy(vs_v, vs_hbm.at[pl.ds(base, per_w)])

  return k(lval_all)


def _sc_row_gather(vsel, cfg16, n_pad, d, chunk):
  rows_per_w = n_pad // _NW
  n_chunks = rows_per_w // chunk
  mesh = plsc.VectorSubcoreMesh(core_axis_name="c", subcore_axis_name="s")

  @functools.partial(
      pl.kernel,
      out_type=jax.ShapeDtypeStruct((n_pad, d), jnp.int32),
      mesh=mesh,
      scratch_types=[
          pltpu.VMEM((rows_per_w,), jnp.int32),
          pltpu.VMEM((chunk, d), jnp.int32),
          pltpu.VMEM((chunk, d), jnp.int32),
          pltpu.SemaphoreType.DMA,
          pltpu.SemaphoreType.DMA,
      ],
  )
  def k(vsel_hbm, cfg_hbm, out_hbm, idx_v, buf0, buf1, sem0, sem1):
    wid = lax.axis_index("s") * _NC + lax.axis_index("c")
    base = wid * rows_per_w
    pltpu.sync_copy(vsel_hbm.at[pl.ds(base, rows_per_w)], idx_v)

    def gather_chunk(c, buf, sem):
      pltpu.async_copy(cfg_hbm.at[idx_v.at[pl.ds(c * chunk, chunk)]], buf, sem)

    gather_chunk(0, buf0, sem0)
    for c in range(n_chunks):
      cur, csem = (buf0, sem0) if c % 2 == 0 else (buf1, sem1)
      nxt, nsem = (buf1, sem1) if c % 2 == 0 else (buf0, sem0)
      if c + 1 < n_chunks:
        gather_chunk(c + 1, nxt, nsem)
      pltpu.make_async_copy(
          cfg_hbm.at[idx_v.at[pl.ds(c * chunk, chunk)]], cur, csem).wait()
      pltpu.sync_copy(cur, out_hbm.at[pl.ds(base + c * chunk, chunk)])

  return k(vsel, cfg16)


def _tc_mlp_body(prev_ref, upd_ref, wp_ref, wu_ref, wg1_ref, wg2_ref,
                 bu_ref, bg_ref, out_ref):
  prev = prev_ref[...]
  upd16 = upd_ref[...]
  proj = jnp.maximum(
      jnp.dot(upd16, wu_ref[...], preferred_element_type=jnp.float32)
      + bu_ref[...], 0.0)
  z = (jnp.dot(prev.astype(jnp.bfloat16), wg1_ref[...],
               preferred_element_type=jnp.float32)
       + jnp.dot(proj.astype(jnp.bfloat16), wg2_ref[...],
                 preferred_element_type=jnp.float32)
       + bg_ref[...])
  gate = jax.nn.sigmoid(z)
  newr = gate * prev + (1.0 - gate) * proj
  out_ref[...] = jnp.where(wp_ref[...] >= 0, newr, prev)


def _tc_mlp(prev_table, upd_sel, winpos2d, wu, wg1, wg2, bu, bg, blk):
  n, d = prev_table.shape
  grid = (n // blk,)
  return pl.pallas_call(
      _tc_mlp_body,
      grid=grid,
      in_specs=[
          pl.BlockSpec((blk, d), lambda i: (i, 0)),
          pl.BlockSpec((blk, d), lambda i: (i, 0)),
          pl.BlockSpec((blk, 1), lambda i: (i, 0)),
          pl.BlockSpec((d, d), lambda i: (0, 0)),
          pl.BlockSpec((d, d), lambda i: (0, 0)),
          pl.BlockSpec((d, d), lambda i: (0, 0)),
          pl.BlockSpec((1, d), lambda i: (0, 0)),
          pl.BlockSpec((1, d), lambda i: (0, 0)),
      ],
      out_specs=pl.BlockSpec((blk, d), lambda i: (i, 0)),
      out_shape=jax.ShapeDtypeStruct((n, d), jnp.float32),
  )(prev_table, upd_sel, winpos2d, wu, wg1, wg2, bu, bg)


def kernel(previous_ast_nodes_encodings, new_cfg_nodes_encodings, key_indices,
           value_indices, W_update, b_update, W_gate, b_gate):
  n_ast, d = previous_ast_nodes_encodings.shape
  n_cfg = new_cfg_nodes_encodings.shape[0]
  e = key_indices.shape[0]

  key_indices = key_indices.astype(jnp.int32)
  value_indices = value_indices.astype(jnp.int32)

  n_pad = 102400
  e_pad = 200704

  pad_e = e_pad - e
  keys_pad = jnp.concatenate(
      [key_indices, n_ast + (jnp.arange(pad_e, dtype=jnp.int32) % 96)])
  vi_pad = jnp.concatenate(
      [value_indices, jnp.arange(pad_e, dtype=jnp.int32) % n_cfg])

  cfg16 = new_cfg_nodes_encodings.astype(jnp.bfloat16)
  cfg_pack = jax.lax.bitcast_convert_type(
      cfg16.reshape(n_cfg, d // 2, 2), jnp.int32)

  lval_all = _sc_winpos_phase1(keys_pad, vi_pad, n_pad, e_pad)
  winpos_p, vsel_p = _sc_winpos_phase2(lval_all, n_pad, n_cfg)
  upd_pack = _sc_row_gather(vsel_p, cfg_pack, n_pad, d // 2, chunk=160)
  upd_sel = jax.lax.bitcast_convert_type(
      upd_pack, jnp.bfloat16).reshape(n_pad, d)

  wg1 = W_gate[:d].astype(jnp.bfloat16)
  wg2 = W_gate[d:].astype(jnp.bfloat16)
  bu = b_update.reshape(1, d)
  bg = b_gate.reshape(1, d)
  winpos2d = winpos_p.reshape(n_pad, 1)

  out = _tc_mlp(previous_ast_nodes_encodings, upd_sel, winpos2d,
                W_update.astype(jnp.bfloat16), wg1, wg2, bu, bg, blk=800)
  return out


# speedup vs baseline: 3.7659x; 3.7659x over previous
import functools

import jax
import jax.numpy as jnp
from jax import lax
from jax.experimental import pallas as pl
from jax.experimental.pallas import tpu as pltpu
from jax.experimental.pallas import tpu_sc as plsc

_NC = 2
_NS = 16
_NW = _NC * _NS


def _sc_winpos_phase1(keys_pad, vals_pad, n_rows_pad, e_pad):
  per_w = e_pad // _NW
  nvec = per_w // 16
  mesh = plsc.VectorSubcoreMesh(core_axis_name="c", subcore_axis_name="s")

  slice_w = n_rows_pad // _NW

  @functools.partial(
      pl.kernel,
      out_type=jax.ShapeDtypeStruct((_NW, _NW, slice_w), jnp.int32),
      mesh=mesh,
      compiler_params=pltpu.CompilerParams(needs_layout_passes=False),
      scratch_types=[
          pltpu.VMEM((per_w,), jnp.int32),
          pltpu.VMEM((per_w,), jnp.int32),
          pltpu.VMEM((n_rows_pad,), jnp.int32),
          pltpu.SemaphoreType.DMA,
      ],
  )
  def k(keys_hbm, vals_hbm, out_hbm, keys_v, vals_v, lval, osem):
    wid = lax.axis_index("s") * _NC + lax.axis_index("c")
    base = wid * per_w
    pltpu.sync_copy(keys_hbm.at[pl.ds(base, per_w)], keys_v)
    pltpu.sync_copy(vals_hbm.at[pl.ds(base, per_w)], vals_v)

    minus1 = jnp.full((16,), -1, jnp.int32)

    def init_body(i, carry):
      for j in range(8):
        lval[pl.ds((i * 8 + j) * 16, 16)] = minus1
      return carry

    lax.fori_loop(0, n_rows_pad // (16 * 8), init_body, 0)

    lane = lax.iota(jnp.int32, 16)

    def scat_body(i, carry):
      keys = keys_v[pl.ds(i * 16, 16)]
      vals = vals_v[pl.ds(i * 16, 16)]
      for l in range(16):
        plsc.store_scatter(lval, [keys], vals, mask=lane == l)
      return carry

    lax.fori_loop(0, nvec, scat_body, 0)
    for s in range(_NW):
      pltpu.async_copy(lval.at[pl.ds(s * slice_w, slice_w)],
                       out_hbm.at[s, wid], osem)
    for s in range(_NW):
      pltpu.make_async_copy(lval.at[pl.ds(s * slice_w, slice_w)],
                            out_hbm.at[s, wid], osem).wait()

  return k(keys_pad, vals_pad)


def _sc_winpos_phase2(lval_all, n_rows_pad, n_cfg):
  per_w = n_rows_pad // _NW
  nv = per_w // 16
  mesh = plsc.VectorSubcoreMesh(core_axis_name="c", subcore_axis_name="s")

  @functools.partial(
      pl.kernel,
      out_type=(jax.ShapeDtypeStruct((n_rows_pad,), jnp.int32),
                jax.ShapeDtypeStruct((n_rows_pad,), jnp.int32)),
      mesh=mesh,
      scratch_types=[
          pltpu.VMEM((_NW, per_w), jnp.int32),
          pltpu.VMEM((per_w,), jnp.int32),
          pltpu.VMEM((per_w,), jnp.int32),
      ],
  )
  def k(lval_hbm, wp_hbm, vs_hbm, tbl_v, wp_v, vs_v):
    wid = lax.axis_index("s") * _NC + lax.axis_index("c")
    base = wid * per_w
    pltpu.sync_copy(lval_hbm.at[wid], tbl_v)

    lane = lax.iota(jnp.int32, 16)
    dummy_mask = 32767
    assert n_cfg > dummy_mask

    def max_body(i, carry):
      acc = tbl_v[0, pl.ds(i * 16, 16)]
      for t in range(1, _NW):
        cur = tbl_v[t, pl.ds(i * 16, 16)]
        acc = jnp.where(cur >= 0, cur, acc)
      wp_v[pl.ds(i * 16, 16)] = acc
      rows = base + i * 16 + lane
      vs_v[pl.ds(i * 16, 16)] = jnp.where(acc >= 0, acc, rows & dummy_mask)
      return carry

    lax.fori_loop(0, nv, max_body, 0)
    pltpu.sync_copy(wp_v, wp_hbm.at[pl.ds(base, per_w)])
    pltpu.sync_copy(vs_v, vs_hbm.at[pl.ds(base, per_w)])

  return k(lval_all)


def _sc_row_gather(vsel, cfg16, n_pad, d, chunk):
  rows_per_w = n_pad // _NW
  n_chunks = rows_per_w // chunk
  mesh = plsc.VectorSubcoreMesh(core_axis_name="c", subcore_axis_name="s")

  @functools.partial(
      pl.kernel,
      out_type=jax.ShapeDtypeStruct((n_pad, d), jnp.int32),
      mesh=mesh,
      scratch_types=[
          pltpu.VMEM((rows_per_w,), jnp.int32),
          pltpu.VMEM((chunk, d), jnp.int32),
          pltpu.VMEM((chunk, d), jnp.int32),
          pltpu.SemaphoreType.DMA,
          pltpu.SemaphoreType.DMA,
      ],
  )
  def k(vsel_hbm, cfg_hbm, out_hbm, idx_v, buf0, buf1, sem0, sem1):
    wid = lax.axis_index("s") * _NC + lax.axis_index("c")
    base = wid * rows_per_w
    pltpu.sync_copy(vsel_hbm.at[pl.ds(base, rows_per_w)], idx_v)

    def gather_chunk(c, buf, sem):
      pltpu.async_copy(cfg_hbm.at[idx_v.at[pl.ds(c * chunk, chunk)]], buf, sem)

    gather_chunk(0, buf0, sem0)
    for c in range(n_chunks):
      cur, csem = (buf0, sem0) if c % 2 == 0 else (buf1, sem1)
      nxt, nsem = (buf1, sem1) if c % 2 == 0 else (buf0, sem0)
      if c + 1 < n_chunks:
        gather_chunk(c + 1, nxt, nsem)
      pltpu.make_async_copy(
          cfg_hbm.at[idx_v.at[pl.ds(c * chunk, chunk)]], cur, csem).wait()
      pltpu.sync_copy(cur, out_hbm.at[pl.ds(base + c * chunk, chunk)])

  return k(vsel, cfg16)


def _tc_pack_body(x_ref, out_ref):
  xb = jax.lax.bitcast_convert_type(x_ref[...], jnp.int32)
  rne = xb + 0x7FFF + (jax.lax.shift_right_logical(xb, 16) & 1)
  h = x_ref.shape[1] // 2
  lo = jax.lax.shift_right_logical(rne[:, :h], 16)
  hi = rne[:, h:] & jnp.int32(-65536)
  out_ref[...] = lo | hi


def _tc_pack_bf16(x, blk):
  n, d = x.shape
  return pl.pallas_call(
      _tc_pack_body,
      grid=(n // blk,),
      in_specs=[pl.BlockSpec((blk, d), lambda i: (i, 0))],
      out_specs=pl.BlockSpec((blk, d // 2), lambda i: (i, 0)),
      out_shape=jax.ShapeDtypeStruct((n, d // 2), jnp.int32),
  )(x)


def _tc_mlp_body(prev_ref, upd_ref, wp_ref, wu_ref, wg1_ref, wg2_ref,
                 bu_ref, bg_ref, out_ref):
  prev = prev_ref[...]
  u = upd_ref[...]
  lo_f = jax.lax.bitcast_convert_type(
      jax.lax.shift_left(u, 16), jnp.float32)
  hi_f = jax.lax.bitcast_convert_type(u & jnp.int32(-65536), jnp.float32)
  upd = jnp.concatenate([lo_f, hi_f], axis=1)
  proj = jnp.maximum(
      jnp.dot(upd, wu_ref[...], preferred_element_type=jnp.float32)
      + bu_ref[...], 0.0)
  z = (jnp.dot(prev, wg1_ref[...], preferred_element_type=jnp.float32)
       + jnp.dot(proj, wg2_ref[...], preferred_element_type=jnp.float32)
       + bg_ref[...])
  gate = jax.nn.sigmoid(z)
  newr = gate * prev + (1.0 - gate) * proj
  out_ref[...] = jnp.where(wp_ref[...] >= 0, newr, prev)


def _tc_mlp(prev_table, upd_sel, winpos2d, wu, wg1, wg2, bu, bg, blk):
  n, d = prev_table.shape
  grid = (n // blk,)
  return pl.pallas_call(
      _tc_mlp_body,
      grid=grid,
      in_specs=[
          pl.BlockSpec((blk, d), lambda i: (i, 0)),
          pl.BlockSpec((blk, d // 2), lambda i: (i, 0)),
          pl.BlockSpec((blk, 1), lambda i: (i, 0)),
          pl.BlockSpec((d, d), lambda i: (0, 0)),
          pl.BlockSpec((d, d), lambda i: (0, 0)),
          pl.BlockSpec((d, d), lambda i: (0, 0)),
          pl.BlockSpec((1, d), lambda i: (0, 0)),
          pl.BlockSpec((1, d), lambda i: (0, 0)),
      ],
      out_specs=pl.BlockSpec((blk, d), lambda i: (i, 0)),
      out_shape=jax.ShapeDtypeStruct((n, d), jnp.float32),
  )(prev_table, upd_sel, winpos2d, wu, wg1, wg2, bu, bg)


def kernel(previous_ast_nodes_encodings, new_cfg_nodes_encodings, key_indices,
           value_indices, W_update, b_update, W_gate, b_gate):
  n_ast, d = previous_ast_nodes_encodings.shape
  n_cfg = new_cfg_nodes_encodings.shape[0]
  e = key_indices.shape[0]

  key_indices = key_indices.astype(jnp.int32)
  value_indices = value_indices.astype(jnp.int32)

  n_pad = 102400
  e_pad = 200704

  pad_e = e_pad - e
  keys_pad = jnp.concatenate(
      [key_indices, n_ast + (jnp.arange(pad_e, dtype=jnp.int32) % 96)])
  vi_pad = jnp.concatenate(
      [value_indices, jnp.arange(pad_e, dtype=jnp.int32) % n_cfg])

  cfg_pack = _tc_pack_bf16(new_cfg_nodes_encodings, blk=1000)

  lval_all = _sc_winpos_phase1(keys_pad, vi_pad, n_pad, e_pad)
  winpos_p, vsel_p = _sc_winpos_phase2(lval_all, n_pad, n_cfg)
  upd_pack = _sc_row_gather(vsel_p, cfg_pack, n_pad, d // 2, chunk=160)

  wg1 = W_gate[:d]
  wg2 = W_gate[d:]
  bu = b_update.reshape(1, d)
  bg = b_gate.reshape(1, d)
  winpos2d = winpos_p.reshape(n_pad, 1)

  out = _tc_mlp(previous_ast_nodes_encodings, upd_pack, winpos2d,
                W_update, wg1, wg2, bu, bg, blk=800)
  return out
